# Initial kernel scaffold; baseline (speedup 1.0000x reference)
import jax
import jax.numpy as jnp
from jax.experimental import pallas as pl


def kernel(x, w1, b1, w2, b2, w3, b3, w4, b4, w5, b5):
    raise NotImplementedError("write your pallas kernel here")


# speedup vs baseline: 2.4289x; 2.4289x over previous
import functools

import jax
import jax.numpy as jnp
from jax.experimental import pallas as pl
from jax.experimental.pallas import tpu as pltpu


def _mlp_softmax_kernel(x_ref,
                        w1_ref, b1_ref, w2_ref, b2_ref, w3_ref, b3_ref,
                        w4_ref, b4_ref, w5_ref, b5_ref,
                        o_ref):
    h = x_ref[...]

    def lin_relu(h, w_ref, b_ref):
        y = jnp.dot(h, w_ref[...], preferred_element_type=jnp.float32) + b_ref[...]
        return jnp.maximum(y, 0.0)

    h = lin_relu(h, w1_ref, b1_ref)
    h = lin_relu(h, w2_ref, b2_ref)
    h = lin_relu(h, w3_ref, b3_ref)
    h = lin_relu(h, w4_ref, b4_ref)
    logits = (jnp.dot(h, w5_ref[...], preferred_element_type=jnp.float32)
              + b5_ref[...])

    m = jnp.max(logits, axis=1, keepdims=True)
    e = jnp.exp(logits - m)
    denom = jnp.sum(e, axis=1, keepdims=True)
    o_ref[...] = (e * pl.reciprocal(denom, approx=False)).astype(o_ref.dtype)


@functools.partial(jax.jit, static_argnames=("block_b",))
def _forward(x, w1, b1, w2, b2, w3, b3, w4, b4, w5, b5, *, block_b=2048):
    B, D_in = x.shape
    out_dim = w5.shape[0]

    block_b = min(block_b, max(256, ((B + 255) // 256) * 256))
    B_pad = ((B + block_b - 1) // block_b) * block_b
    if B_pad != B:
        x = jnp.pad(x, ((0, B_pad - B), (0, 0)))

    wts = [w.T for w in (w1, w2, w3, w4, w5)]
    bts = [b.reshape(1, -1) for b in (b1, b2, b3, b4, b5)]

    def full_spec(shape):
        return pl.BlockSpec(shape, lambda i: (0, 0))

    grid = (B_pad // block_b,)

    flops = 2 * B_pad * (D_in * 10 + 10 * 50 + 50 * 10 + 10 * 5 + 5 * out_dim)
    param_bytes = sum(int(v.size) * 4 for v in wts + bts)
    bytes_accessed = B_pad * (D_in + out_dim) * 4 + param_bytes

    in_specs = [pl.BlockSpec((block_b, D_in), lambda i: (i, 0))]
    operands = [x]
    for w, b in zip(wts, bts):
        in_specs += [full_spec(w.shape), full_spec(b.shape)]
        operands += [w, b]

    out = pl.pallas_call(
        _mlp_softmax_kernel,
        out_shape=jax.ShapeDtypeStruct((B_pad, out_dim), jnp.float32),
        grid_spec=pltpu.PrefetchScalarGridSpec(
            num_scalar_prefetch=0,
            grid=grid,
            in_specs=in_specs,
            out_specs=pl.BlockSpec((block_b, out_dim), lambda i: (i, 0)),
        ),
        compiler_params=pltpu.CompilerParams(
            dimension_semantics=("parallel",),
        ),
        cost_estimate=pl.CostEstimate(
            flops=flops,
            transcendentals=B_pad * out_dim,
            bytes_accessed=bytes_accessed),
    )(*operands)

    return out[:B]


def kernel(x, w1, b1, w2, b2, w3, b3, w4, b4, w5, b5):
    return _forward(x, w1, b1, w2, b2, w3, b3, w4, b4, w5, b5)


# speedup vs baseline: 2.4476x; 1.0077x over previous
import functools

import jax
import jax.numpy as jnp
from jax.experimental import pallas as pl
from jax.experimental.pallas import tpu as pltpu


def _mlp_softmax_kernel(x_ref,
                        w1_ref, b1_ref, w2_ref, b2_ref, w3_ref, b3_ref,
                        w4_ref, b4_ref, w5_ref, b5_ref,
                        o_ref):
    h = jax.lax.dot_general(
        w1_ref[...], x_ref[...], (((1,), (1,)), ((), ())),
        preferred_element_type=jnp.float32)
    h = jnp.maximum(h + b1_ref[...], 0.0)

    def lin_relu(w_ref, b_ref, h):
        y = jnp.dot(w_ref[...], h, preferred_element_type=jnp.float32) + b_ref[...]
        return jnp.maximum(y, 0.0)

    h = lin_relu(w2_ref, b2_ref, h)
    h = lin_relu(w3_ref, b3_ref, h)
    h = lin_relu(w4_ref, b4_ref, h)

    logits = jax.lax.dot_general(
        h, w5_ref[...], (((0,), (1,)), ((), ())),
        preferred_element_type=jnp.float32) + b5_ref[...]

    m = jnp.max(logits, axis=1, keepdims=True)
    e = jnp.exp(logits - m)
    denom = jnp.sum(e, axis=1, keepdims=True)
    o_ref[...] = (e * pl.reciprocal(denom, approx=False)).astype(o_ref.dtype)


@functools.partial(jax.jit, static_argnames=("block_b",))
def _forward(x, w1, b1, w2, b2, w3, b3, w4, b4, w5, b5, *, block_b=2048):
    B, D_in = x.shape
    out_dim = w5.shape[0]

    block_b = min(block_b, max(256, ((B + 255) // 256) * 256))
    B_pad = ((B + block_b - 1) // block_b) * block_b
    if B_pad != B:
        x = jnp.pad(x, ((0, B_pad - B), (0, 0)))

    bcol = [b.reshape(-1, 1) for b in (b1, b2, b3, b4)]
    b5r = b5.reshape(1, -1)

    def full_spec(shape):
        return pl.BlockSpec(shape, lambda i: (0, 0))

    grid = (B_pad // block_b,)

    flops = 2 * B_pad * (D_in * 10 + 10 * 50 + 50 * 10 + 10 * 5 + 5 * out_dim)
    param_bytes = sum(int(v.size) * 4
                      for v in (w1, w2, w3, w4, w5, b1, b2, b3, b4, b5))
    bytes_accessed = B_pad * (D_in + out_dim) * 4 + param_bytes

    operands = [x,
                w1, bcol[0], w2, bcol[1], w3, bcol[2], w4, bcol[3], w5, b5r]
    in_specs = [pl.BlockSpec((block_b, D_in), lambda i: (i, 0))]
    for v in operands[1:]:
        in_specs.append(full_spec(v.shape))

    out = pl.pallas_call(
        _mlp_softmax_kernel,
        out_shape=jax.ShapeDtypeStruct((B_pad, out_dim), jnp.float32),
        grid_spec=pltpu.PrefetchScalarGridSpec(
            num_scalar_prefetch=0,
            grid=grid,
            in_specs=in_specs,
            out_specs=pl.BlockSpec((block_b, out_dim), lambda i: (i, 0)),
        ),
        compiler_params=pltpu.CompilerParams(
            dimension_semantics=("parallel",),
        ),
        cost_estimate=pl.CostEstimate(
            flops=flops,
            transcendentals=B_pad * out_dim,
            bytes_accessed=bytes_accessed),
    )(*operands)

    return out[:B]


def kernel(x, w1, b1, w2, b2, w3, b3, w4, b4, w5, b5):
    return _forward(x, w1, b1, w2, b2, w3, b3, w4, b4, w5, b5)


# speedup vs baseline: 2.9446x; 1.2031x over previous
import functools

import jax
import jax.numpy as jnp
from jax.experimental import pallas as pl
from jax.experimental.pallas import tpu as pltpu


def _mlp_softmax_kernel(x_ref,
                        w1_ref, b1_ref, w2_ref, b2_ref, w3_ref, b3_ref,
                        w4_ref, b4_ref, w5_ref, b5_ref,
                        o_ref):
    h = jax.lax.dot_general(
        w1_ref[...], x_ref[...], (((1,), (1,)), ((), ())),
        preferred_element_type=jnp.float32)
    h = jnp.maximum(h + b1_ref[...], 0.0)

    def lin_relu(w_ref, b_ref, h):
        y = jnp.dot(w_ref[...], h, preferred_element_type=jnp.float32) + b_ref[...]
        return jnp.maximum(y, 0.0)

    h = lin_relu(w2_ref, b2_ref, h)
    h = lin_relu(w3_ref, b3_ref, h)
    h = lin_relu(w4_ref, b4_ref, h)

    logits = jax.lax.dot_general(
        h, w5_ref[...], (((0,), (1,)), ((), ())),
        preferred_element_type=jnp.float32) + b5_ref[...]

    m = jnp.max(logits, axis=1, keepdims=True)
    e = jnp.exp(logits - m)
    denom = jnp.sum(e, axis=1, keepdims=True)
    o_ref[...] = (e * pl.reciprocal(denom, approx=False)).astype(o_ref.dtype)


@functools.partial(jax.jit, static_argnames=("block_b",))
def _forward(x, w1, b1, w2, b2, w3, b3, w4, b4, w5, b5, *, block_b=4096):
    B, D_in = x.shape
    out_dim = w5.shape[0]

    block_b = min(block_b, max(256, ((B + 255) // 256) * 256))
    B_pad = ((B + block_b - 1) // block_b) * block_b
    if B_pad != B:
        x = jnp.pad(x, ((0, B_pad - B), (0, 0)))

    bcol = [b.reshape(-1, 1) for b in (b1, b2, b3, b4)]
    b5r = b5.reshape(1, -1)

    def full_spec(shape):
        return pl.BlockSpec(shape, lambda i: (0, 0))

    grid = (B_pad // block_b,)

    flops = 2 * B_pad * (D_in * 10 + 10 * 50 + 50 * 10 + 10 * 5 + 5 * out_dim)
    param_bytes = sum(int(v.size) * 4
                      for v in (w1, w2, w3, w4, w5, b1, b2, b3, b4, b5))
    bytes_accessed = B_pad * (D_in + out_dim) * 4 + param_bytes

    operands = [x,
                w1, bcol[0], w2, bcol[1], w3, bcol[2], w4, bcol[3], w5, b5r]
    in_specs = [pl.BlockSpec((block_b, D_in), lambda i: (i, 0))]
    for v in operands[1:]:
        in_specs.append(full_spec(v.shape))

    out = pl.pallas_call(
        _mlp_softmax_kernel,
        out_shape=jax.ShapeDtypeStruct((B_pad, out_dim), jnp.float32),
        grid_spec=pltpu.PrefetchScalarGridSpec(
            num_scalar_prefetch=0,
            grid=grid,
            in_specs=in_specs,
            out_specs=pl.BlockSpec((block_b, out_dim), lambda i: (i, 0)),
        ),
        compiler_params=pltpu.CompilerParams(
            dimension_semantics=("parallel",),
        ),
        cost_estimate=pl.CostEstimate(
            flops=flops,
            transcendentals=B_pad * out_dim,
            bytes_accessed=bytes_accessed),
    )(*operands)

    return out[:B]


def kernel(x, w1, b1, w2, b2, w3, b3, w4, b4, w5, b5):
    return _forward(x, w1, b1, w2, b2, w3, b3, w4, b4, w5, b5)


# speedup vs baseline: 3.1480x; 1.0691x over previous
import functools

import jax
import jax.numpy as jnp
from jax.experimental import pallas as pl
from jax.experimental.pallas import tpu as pltpu


def _mlp_softmax_kernel(x_ref,
                        w1_ref, b1_ref, w2_ref, b2_ref, w3_ref, b3_ref,
                        w4_ref, b4_ref, w5_ref, b5_ref,
                        o_ref):
    h = jax.lax.dot_general(
        w1_ref[...], x_ref[...], (((1,), (1,)), ((), ())),
        preferred_element_type=jnp.float32)
    h = jnp.maximum(h + b1_ref[...], 0.0)

    def lin_relu(w_ref, b_ref, h):
        y = jnp.dot(w_ref[...], h, preferred_element_type=jnp.float32) + b_ref[...]
        return jnp.maximum(y, 0.0)

    h = lin_relu(w2_ref, b2_ref, h)
    h = lin_relu(w3_ref, b3_ref, h)
    h = lin_relu(w4_ref, b4_ref, h)

    logits = jax.lax.dot_general(
        h, w5_ref[...], (((0,), (1,)), ((), ())),
        preferred_element_type=jnp.float32) + b5_ref[...]

    m = jnp.max(logits, axis=1, keepdims=True)
    e = jnp.exp(logits - m)
    denom = jnp.sum(e, axis=1, keepdims=True)
    o_ref[...] = (e * pl.reciprocal(denom, approx=False)).astype(o_ref.dtype)


@functools.partial(jax.jit, static_argnames=("block_b",))
def _forward(x, w1, b1, w2, b2, w3, b3, w4, b4, w5, b5, *, block_b=8192):
    B, D_in = x.shape
    out_dim = w5.shape[0]

    block_b = min(block_b, max(256, ((B + 255) // 256) * 256))
    B_pad = ((B + block_b - 1) // block_b) * block_b
    if B_pad != B:
        x = jnp.pad(x, ((0, B_pad - B), (0, 0)))

    bcol = [b.reshape(-1, 1) for b in (b1, b2, b3, b4)]
    b5r = b5.reshape(1, -1)

    def full_spec(shape):
        return pl.BlockSpec(shape, lambda i: (0, 0))

    grid = (B_pad // block_b,)

    flops = 2 * B_pad * (D_in * 10 + 10 * 50 + 50 * 10 + 10 * 5 + 5 * out_dim)
    param_bytes = sum(int(v.size) * 4
                      for v in (w1, w2, w3, w4, w5, b1, b2, b3, b4, b5))
    bytes_accessed = B_pad * (D_in + out_dim) * 4 + param_bytes

    operands = [x,
                w1, bcol[0], w2, bcol[1], w3, bcol[2], w4, bcol[3], w5, b5r]
    in_specs = [pl.BlockSpec((block_b, D_in), lambda i: (i, 0))]
    for v in operands[1:]:
        in_specs.append(full_spec(v.shape))

    out = pl.pallas_call(
        _mlp_softmax_kernel,
        out_shape=jax.ShapeDtypeStruct((B_pad, out_dim), jnp.float32),
        grid_spec=pltpu.PrefetchScalarGridSpec(
            num_scalar_prefetch=0,
            grid=grid,
            in_specs=in_specs,
            out_specs=pl.BlockSpec((block_b, out_dim), lambda i: (i, 0)),
        ),
        compiler_params=pltpu.CompilerParams(
            dimension_semantics=("parallel",),
        ),
        cost_estimate=pl.CostEstimate(
            flops=flops,
            transcendentals=B_pad * out_dim,
            bytes_accessed=bytes_accessed),
    )(*operands)

    return out[:B]


def kernel(x, w1, b1, w2, b2, w3, b3, w4, b4, w5, b5):
    return _forward(x, w1, b1, w2, b2, w3, b3, w4, b4, w5, b5)
